# baseline (speedup 1.0000x reference)
import functools

import jax
import jax.numpy as jnp
from jax import lax
from jax.experimental import pallas as pl
from jax.experimental.pallas import tpu as pltpu
from jax.experimental.pallas import tpu_sc as plsc

NC, NS, L = 2, 16, 16
NW = NC * NS
BINS = 512
CH = 1024
IB = 128
TBLK = 8192
NSLAB = 4


def _minmax_body(x_ref, min_ref, max_ref, sm, sM):
    i = pl.program_id(0)
    x = x_ref[...]
    bmin = jnp.min(x, axis=1, keepdims=True)
    bmax = jnp.max(x, axis=1, keepdims=True)

    @pl.when(i == 0)
    def _():
        sm[...] = jnp.broadcast_to(bmin, sm.shape)
        sM[...] = jnp.broadcast_to(bmax, sM.shape)

    @pl.when(i > 0)
    def _():
        sm[...] = jnp.minimum(sm[...], jnp.broadcast_to(bmin, sm.shape))
        sM[...] = jnp.maximum(sM[...], jnp.broadcast_to(bmax, sM.shape))

    @pl.when(i == pl.num_programs(0) - 1)
    def _():
        min_ref[...] = sm[...]
        max_ref[...] = sM[...]


def _coord_minmax(coords_t):
    n = coords_t.shape[1]
    blk = 65536
    return pl.pallas_call(
        _minmax_body,
        grid=(n // blk,),
        in_specs=[pl.BlockSpec((3, blk), lambda i: (0, i))],
        out_specs=[pl.BlockSpec((3, 128), lambda i: (0, 0))] * 2,
        out_shape=[jax.ShapeDtypeStruct((3, 128), jnp.float32)] * 2,
        scratch_shapes=[pltpu.VMEM((3, 128), jnp.float32)] * 2,
    )(coords_t)


def _transpose_body(xt_ref, out_ref):
    y = xt_ref[...].T
    out_ref[...] = jnp.concatenate([y, jnp.zeros_like(y)], axis=1)


def _transpose_slab(feat_t, slab, slab_n):
    d, n = feat_t.shape
    blocks = slab_n // TBLK
    return pl.pallas_call(
        _transpose_body,
        grid=(blocks,),
        in_specs=[pl.BlockSpec((d, TBLK),
                               lambda i, s=slab, b=blocks: (0, s * b + i))],
        out_specs=pl.BlockSpec((TBLK, 2 * d), lambda i: (i, 0)),
        out_shape=jax.ShapeDtypeStruct((slab_n, 2 * d), jnp.float32),
    )(feat_t)


def _bin_body(params_h, coords_h, zeros16_h, ones_h,
              binid_h, binid1_h, cnts_h,
              params_v, xv, yv, zv, bins_v, bins1_v, ones_v,
              cnt_sh):
    cid = lax.axis_index("c")
    sid = lax.axis_index("s")
    n_pts = coords_h.shape[1]
    pt = n_pts // NW
    base = (cid * NS + sid) * pt

    pltpu.sync_copy(params_h, params_v)
    pltpu.sync_copy(ones_h, ones_v)

    @pl.when(sid == 0)
    def _():
        pltpu.sync_copy(zeros16_h, cnt_sh)

    plsc.subcore_barrier()

    minx, miny, minz = params_v[0], params_v[1], params_v[2]
    exx, exy, exz = params_v[3], params_v[4], params_v[5]
    nxf, nyf, nzf = params_v[6], params_v[7], params_v[8]
    nyi = nyf.astype(jnp.int32)
    nzi = nzf.astype(jnp.int32)
    mxx = nxf.astype(jnp.int32) - 1
    mxy = nyi - 1
    mxz = nzi - 1

    def chunk(i, carry):
        p0 = pl.multiple_of(base + i * CH, CH)
        pltpu.sync_copy(coords_h.at[0, pl.ds(p0, CH)], xv)
        pltpu.sync_copy(coords_h.at[1, pl.ds(p0, CH)], yv)
        pltpu.sync_copy(coords_h.at[2, pl.ds(p0, CH)], zv)
        for g in range(CH // L):
            x = xv[pl.ds(g * L, L)]
            y = yv[pl.ds(g * L, L)]
            z = zv[pl.ds(g * L, L)]
            ix = jnp.minimum(jnp.maximum(((x - minx) / exx * nxf).astype(jnp.int32), 0), mxx)
            iy = jnp.minimum(jnp.maximum(((y - miny) / exy * nyf).astype(jnp.int32), 0), mxy)
            iz = jnp.minimum(jnp.maximum(((z - minz) / exz * nzf).astype(jnp.int32), 0), mxz)
            b = (ix * nyi + iy) * nzi + iz
            bins_v[g // 8, pl.ds((g % 8) * L, L)] = b
            bins1_v[pl.ds(g * L, L)] = b
        row0 = pl.multiple_of(p0 // 128, CH // 128)
        pltpu.sync_copy(bins_v, binid_h.at[pl.ds(row0, CH // 128)])
        pltpu.sync_copy(bins1_v, binid1_h.at[pl.ds(p0, CH)])
        for j in range(CH // IB):
            pltpu.sync_copy(ones_v, cnt_sh.at[bins_v.at[j]], add=True)
        return carry

    lax.fori_loop(0, pt // CH, chunk, 0)
    plsc.subcore_barrier()

    @pl.when(sid == 0)
    def _():
        pltpu.sync_copy(cnt_sh, cnts_h.at[cid])


def _sc_binning(params, coords_t, zeros16, ones):
    n = coords_t.shape[1]
    mesh = plsc.VectorSubcoreMesh(core_axis_name="c", subcore_axis_name="s")
    return pl.kernel(
        _bin_body,
        out_type=[
            jax.ShapeDtypeStruct((n // 128, 128), jnp.int32),
            jax.ShapeDtypeStruct((n,), jnp.int32),
            jax.ShapeDtypeStruct((NC, BINS, 16), jnp.float32),
        ],
        mesh=mesh,
        compiler_params=pltpu.CompilerParams(needs_layout_passes=False,
                                             use_tc_tiling_on_sc=False),
        scratch_types=[
            pltpu.VMEM((16, 16), jnp.float32),
            pltpu.VMEM((CH,), jnp.float32),
            pltpu.VMEM((CH,), jnp.float32),
            pltpu.VMEM((CH,), jnp.float32),
            pltpu.VMEM((CH // 128, 128), jnp.int32),
            pltpu.VMEM((CH,), jnp.int32),
            pltpu.VMEM((IB, 16), jnp.float32),
            pltpu.VMEM_SHARED((BINS, 16), jnp.float32),
        ],
    )(params, coords_t, zeros16, ones)


def _scat_body(feat_h, binid_h, zeros64_h, sums_h,
               feat0_v, feat1_v, bins_v, sem0, sem1, acc_sh,
               *, row_base):
    cid = lax.axis_index("c")
    sid = lax.axis_index("s")
    n_pts = feat_h.shape[0]
    pt = n_pts // NW
    base = (cid * NS + sid) * pt

    @pl.when(sid == 0)
    def _():
        pltpu.sync_copy(zeros64_h, acc_sh)

    plsc.subcore_barrier()

    bufs = (feat0_v, feat1_v)
    sems = (sem0, sem1)
    hch = 256
    nhalves = pt // hch

    p00 = pl.multiple_of(base, hch)
    pltpu.async_copy(feat_h.at[pl.ds(p00, hch)], feat0_v, sem0)

    def half(hc, carry):
        pi = pl.multiple_of(base + hc * hch, hch)
        pn = pl.multiple_of(pi + hch, hch)
        for q in range(4):
            @pl.when(hc % 4 == q)
            def _():
                if q == 0:
                    row0 = pl.multiple_of(row_base + pi // 128, 8)
                    pltpu.sync_copy(binid_h.at[pl.ds(row0, 8)], bins_v)
                b = q % 2
                @pl.when(hc + 1 < nhalves)
                def _():
                    pltpu.async_copy(feat_h.at[pl.ds(pn, hch)],
                                     bufs[1 - b], sems[1 - b])
                pltpu.make_async_copy(feat_h.at[pl.ds(pi, hch)],
                                      bufs[b], sems[b]).wait()
                for j in range(hch // IB):
                    pltpu.sync_copy(bufs[b].at[pl.ds(j * IB, IB)],
                                    acc_sh.at[bins_v.at[q * (hch // IB) + j]],
                                    add=True)
        return carry

    lax.fori_loop(0, nhalves, half, 0)
    plsc.subcore_barrier()

    @pl.when(sid == 0)
    def _():
        pltpu.sync_copy(acc_sh, sums_h.at[cid])


def _sc_scatter(feat_slab, binid, zeros64, slab):
    n = feat_slab.shape[0]
    mesh = plsc.VectorSubcoreMesh(core_axis_name="c", subcore_axis_name="s")
    body = functools.partial(_scat_body, row_base=slab * (n // 128))
    return pl.kernel(
        body,
        out_type=[
            jax.ShapeDtypeStruct((NC, BINS, 128), jnp.float32),
        ],
        mesh=mesh,
        compiler_params=pltpu.CompilerParams(needs_layout_passes=False,
                                             use_tc_tiling_on_sc=False),
        scratch_types=[
            pltpu.VMEM((256, 128), jnp.float32),
            pltpu.VMEM((256, 128), jnp.float32),
            pltpu.VMEM((CH // 128, 128), jnp.int32),
            pltpu.SemaphoreType.DMA,
            pltpu.SemaphoreType.DMA,
            pltpu.VMEM_SHARED((BINS, 128), jnp.float32),
        ],
    )(feat_slab, binid, zeros64)


def _epi_body(s0_ref, s1_ref, s2_ref, s3_ref, cnts_ref,
              proto_ref, cs_ref, mask_ref, cnt_ref):
    sw = (jnp.sum(s0_ref[...], axis=0) + jnp.sum(s1_ref[...], axis=0)
          + jnp.sum(s2_ref[...], axis=0) + jnp.sum(s3_ref[...], axis=0))
    s = sw[:, :64]
    cnt = jnp.sum(cnts_ref[...], axis=0)[:, 0]
    proto = s / jnp.maximum(cnt, 1.0)[:, None]
    proto_ref[...] = proto
    nrm = jnp.sqrt(jnp.sum(proto * proto, axis=1, keepdims=True))
    sn = proto / (nrm + 1e-12)
    cos = lax.dot_general(sn, sn, (((1,), (1,)), ((), ())),
                          preferred_element_type=jnp.float32)
    r = lax.broadcasted_iota(jnp.int32, (BINS, BINS), 0)
    c = lax.broadcasted_iota(jnp.int32, (BINS, BINS), 1)
    cosm = jnp.where(r == c, 0.0, cos)
    cs = jnp.sum(cosm, axis=1)
    cs_ref[...] = cs
    mask_ref[...] = (cs > jnp.mean(cs)).astype(jnp.int32)
    cnt_ref[...] = cnt


def _epilogue(sums_ps, cnts_p):
    return pl.pallas_call(
        _epi_body,
        out_shape=[
            jax.ShapeDtypeStruct((BINS, 64), jnp.float32),
            jax.ShapeDtypeStruct((BINS,), jnp.float32),
            jax.ShapeDtypeStruct((BINS,), jnp.int32),
            jax.ShapeDtypeStruct((BINS,), jnp.float32),
        ],
    )(*sums_ps, cnts_p)


def kernel(feat, coords, n_x, n_y, n_z):
    n, d = feat.shape
    coords_t = coords.T
    feat_t = feat.T
    mins3, maxs3 = _coord_minmax(coords_t)
    mins = mins3[:, 0]
    maxs = maxs3[:, 0]
    extent = maxs - mins + 1e-6
    nf = jnp.stack([jnp.asarray(n_x), jnp.asarray(n_y), jnp.asarray(n_z)]
                   ).astype(jnp.float32)
    vals = jnp.concatenate([mins, extent, nf])
    params = jnp.zeros((16, 16), jnp.float32).at[:9, :].set(vals[:, None])
    zeros64 = jnp.zeros((BINS, 128), jnp.float32)
    zeros16 = jnp.zeros((BINS, 16), jnp.float32)
    ones = jnp.ones((IB, 16), jnp.float32)
    binid2, binid1, cnts_p = _sc_binning(params, coords_t, zeros16, ones)
    slab_n = n // NSLAB
    sums_ps = []
    for s in range(NSLAB):
        feat_slab = _transpose_slab(feat_t, s, slab_n)
        (sp,) = _sc_scatter(feat_slab, binid2, zeros64, s)
        sums_ps.append(sp)
    proto, cs, mask_i, counts = _epilogue(sums_ps, cnts_p)
    return proto, cs, mask_i.astype(bool), binid1, counts


# speedup vs baseline: 1.0971x; 1.0971x over previous
import functools

import jax
import jax.numpy as jnp
from jax import lax
from jax.experimental import pallas as pl
from jax.experimental.pallas import tpu as pltpu
from jax.experimental.pallas import tpu_sc as plsc

NC, NS, L = 2, 16, 16
NW = NC * NS
BINS = 512
CH = 1024
IB = 128
TBLK = 4096
NSLAB = 4


def _minmax_body(x_ref, min_ref, max_ref, xo_ref, yo_ref, zo_ref, sm, sM):
    i = pl.program_id(0)
    x = x_ref[...]
    xo_ref[...] = x[0, :]
    yo_ref[...] = x[1, :]
    zo_ref[...] = x[2, :]
    bmin = jnp.min(x, axis=1, keepdims=True)
    bmax = jnp.max(x, axis=1, keepdims=True)

    @pl.when(i == 0)
    def _():
        sm[...] = jnp.broadcast_to(bmin, sm.shape)
        sM[...] = jnp.broadcast_to(bmax, sM.shape)

    @pl.when(i > 0)
    def _():
        sm[...] = jnp.minimum(sm[...], jnp.broadcast_to(bmin, sm.shape))
        sM[...] = jnp.maximum(sM[...], jnp.broadcast_to(bmax, sM.shape))

    @pl.when(i == pl.num_programs(0) - 1)
    def _():
        min_ref[...] = sm[...]
        max_ref[...] = sM[...]


def _coord_minmax(coords_t):
    n = coords_t.shape[1]
    blk = 65536
    return pl.pallas_call(
        _minmax_body,
        grid=(n // blk,),
        in_specs=[pl.BlockSpec((3, blk), lambda i: (0, i))],
        out_specs=[pl.BlockSpec((3, 128), lambda i: (0, 0))] * 2
        + [pl.BlockSpec((blk,), lambda i: (i,))] * 3,
        out_shape=[jax.ShapeDtypeStruct((3, 128), jnp.float32)] * 2
        + [jax.ShapeDtypeStruct((n,), jnp.float32)] * 3,
        scratch_shapes=[pltpu.VMEM((3, 128), jnp.float32)] * 2,
    )(coords_t)


def _transpose_body(xt_ref, out_ref):
    y = xt_ref[...].T
    out_ref[...] = jnp.concatenate([y, jnp.zeros_like(y)], axis=1)


def _transpose_slab(feat_t, slab, slab_n):
    d, n = feat_t.shape
    blocks = slab_n // TBLK
    return pl.pallas_call(
        _transpose_body,
        grid=(blocks,),
        in_specs=[pl.BlockSpec((d, TBLK),
                               lambda i, s=slab, b=blocks: (0, s * b + i))],
        out_specs=pl.BlockSpec((TBLK, 2 * d), lambda i: (i, 0)),
        out_shape=jax.ShapeDtypeStruct((slab_n, 2 * d), jnp.float32),
    )(feat_t)


def _bin_body(params_h, xc_h, yc_h, zc_h, zeros16_h, ones_h,
              binid_h, binid1_h, cnts_h,
              params_v, xv, yv, zv, bins_v, bins1_v, ones_v,
              cnt_sh):
    cid = lax.axis_index("c")
    sid = lax.axis_index("s")
    n_pts = xc_h.shape[0]
    pt = n_pts // NW
    base = (cid * NS + sid) * pt

    pltpu.sync_copy(params_h, params_v)
    pltpu.sync_copy(ones_h, ones_v)

    @pl.when(sid == 0)
    def _():
        pltpu.sync_copy(zeros16_h, cnt_sh)

    plsc.subcore_barrier()

    minx, miny, minz = params_v[0], params_v[1], params_v[2]
    exx, exy, exz = params_v[3], params_v[4], params_v[5]
    nxf, nyf, nzf = params_v[6], params_v[7], params_v[8]
    nyi = nyf.astype(jnp.int32)
    nzi = nzf.astype(jnp.int32)
    mxx = nxf.astype(jnp.int32) - 1
    mxy = nyi - 1
    mxz = nzi - 1

    def chunk(i, carry):
        p0 = pl.multiple_of(base + i * CH, CH)
        pltpu.sync_copy(xc_h.at[pl.ds(p0, CH)], xv)
        pltpu.sync_copy(yc_h.at[pl.ds(p0, CH)], yv)
        pltpu.sync_copy(zc_h.at[pl.ds(p0, CH)], zv)
        for g in range(CH // L):
            x = xv[pl.ds(g * L, L)]
            y = yv[pl.ds(g * L, L)]
            z = zv[pl.ds(g * L, L)]
            ix = jnp.minimum(jnp.maximum(((x - minx) / exx * nxf).astype(jnp.int32), 0), mxx)
            iy = jnp.minimum(jnp.maximum(((y - miny) / exy * nyf).astype(jnp.int32), 0), mxy)
            iz = jnp.minimum(jnp.maximum(((z - minz) / exz * nzf).astype(jnp.int32), 0), mxz)
            b = (ix * nyi + iy) * nzi + iz
            bins_v[g // 8, pl.ds((g % 8) * L, L)] = b
            bins1_v[pl.ds(g * L, L)] = b
        row0 = pl.multiple_of(p0 // 128, CH // 128)
        pltpu.sync_copy(bins_v, binid_h.at[pl.ds(row0, CH // 128)])
        pltpu.sync_copy(bins1_v, binid1_h.at[pl.ds(p0, CH)])
        for j in range(CH // IB):
            pltpu.sync_copy(ones_v, cnt_sh.at[bins_v.at[j]], add=True)
        return carry

    lax.fori_loop(0, pt // CH, chunk, 0)
    plsc.subcore_barrier()

    @pl.when(sid == 0)
    def _():
        pltpu.sync_copy(cnt_sh, cnts_h.at[cid])


def _sc_binning(params, xc, yc, zc, zeros16, ones):
    n = xc.shape[0]
    mesh = plsc.VectorSubcoreMesh(core_axis_name="c", subcore_axis_name="s")
    return pl.kernel(
        _bin_body,
        out_type=[
            jax.ShapeDtypeStruct((n // 128, 128), jnp.int32),
            jax.ShapeDtypeStruct((n,), jnp.int32),
            jax.ShapeDtypeStruct((NC, BINS, 16), jnp.float32),
        ],
        mesh=mesh,
        compiler_params=pltpu.CompilerParams(needs_layout_passes=False,
                                             use_tc_tiling_on_sc=False),
        scratch_types=[
            pltpu.VMEM((16, 16), jnp.float32),
            pltpu.VMEM((CH,), jnp.float32),
            pltpu.VMEM((CH,), jnp.float32),
            pltpu.VMEM((CH,), jnp.float32),
            pltpu.VMEM((CH // 128, 128), jnp.int32),
            pltpu.VMEM((CH,), jnp.int32),
            pltpu.VMEM((IB, 16), jnp.float32),
            pltpu.VMEM_SHARED((BINS, 16), jnp.float32),
        ],
    )(params, xc, yc, zc, zeros16, ones)


def _scat_body(feat_h, binid_h, zeros64_h, sums_h,
               feat0_v, feat1_v, bins_v, sem0, sem1, acc_sh,
               *, row_base):
    cid = lax.axis_index("c")
    sid = lax.axis_index("s")
    n_pts = feat_h.shape[0]
    pt = n_pts // NW
    base = (cid * NS + sid) * pt

    @pl.when(sid == 0)
    def _():
        pltpu.sync_copy(zeros64_h, acc_sh)

    plsc.subcore_barrier()

    bufs = (feat0_v, feat1_v)
    sems = (sem0, sem1)
    hch = 256
    nhalves = pt // hch

    p00 = pl.multiple_of(base, hch)
    pltpu.async_copy(feat_h.at[pl.ds(p00, hch)], feat0_v, sem0)

    def half(hc, carry):
        pi = pl.multiple_of(base + hc * hch, hch)
        pn = pl.multiple_of(pi + hch, hch)
        for q in range(4):
            @pl.when(hc % 4 == q)
            def _():
                if q == 0:
                    row0 = pl.multiple_of(row_base + pi // 128, 8)
                    pltpu.sync_copy(binid_h.at[pl.ds(row0, 8)], bins_v)
                b = q % 2
                @pl.when(hc + 1 < nhalves)
                def _():
                    pltpu.async_copy(feat_h.at[pl.ds(pn, hch)],
                                     bufs[1 - b], sems[1 - b])
                pltpu.make_async_copy(feat_h.at[pl.ds(pi, hch)],
                                      bufs[b], sems[b]).wait()
                for j in range(hch // IB):
                    pltpu.sync_copy(bufs[b].at[pl.ds(j * IB, IB)],
                                    acc_sh.at[bins_v.at[q * (hch // IB) + j]],
                                    add=True)
        return carry

    lax.fori_loop(0, nhalves, half, 0)
    plsc.subcore_barrier()

    @pl.when(sid == 0)
    def _():
        pltpu.sync_copy(acc_sh, sums_h.at[cid])


def _sc_scatter(feat_slab, binid, zeros64, slab):
    n = feat_slab.shape[0]
    mesh = plsc.VectorSubcoreMesh(core_axis_name="c", subcore_axis_name="s")
    body = functools.partial(_scat_body, row_base=slab * (n // 128))
    return pl.kernel(
        body,
        out_type=[
            jax.ShapeDtypeStruct((NC, BINS, 128), jnp.float32),
        ],
        mesh=mesh,
        compiler_params=pltpu.CompilerParams(needs_layout_passes=False,
                                             use_tc_tiling_on_sc=False),
        scratch_types=[
            pltpu.VMEM((256, 128), jnp.float32),
            pltpu.VMEM((256, 128), jnp.float32),
            pltpu.VMEM((CH // 128, 128), jnp.int32),
            pltpu.SemaphoreType.DMA,
            pltpu.SemaphoreType.DMA,
            pltpu.VMEM_SHARED((BINS, 128), jnp.float32),
        ],
    )(feat_slab, binid, zeros64)


def _epi_body(s0_ref, s1_ref, s2_ref, s3_ref, cnts_ref,
              proto_ref, cs_ref, mask_ref, cnt_ref):
    sw = (jnp.sum(s0_ref[...], axis=0) + jnp.sum(s1_ref[...], axis=0)
          + jnp.sum(s2_ref[...], axis=0) + jnp.sum(s3_ref[...], axis=0))
    s = sw[:, :64]
    cnt = jnp.sum(cnts_ref[...], axis=0)[:, 0]
    proto = s / jnp.maximum(cnt, 1.0)[:, None]
    proto_ref[...] = proto
    nrm = jnp.sqrt(jnp.sum(proto * proto, axis=1, keepdims=True))
    sn = proto / (nrm + 1e-12)
    cos = lax.dot_general(sn, sn, (((1,), (1,)), ((), ())),
                          preferred_element_type=jnp.float32)
    r = lax.broadcasted_iota(jnp.int32, (BINS, BINS), 0)
    c = lax.broadcasted_iota(jnp.int32, (BINS, BINS), 1)
    cosm = jnp.where(r == c, 0.0, cos)
    cs = jnp.sum(cosm, axis=1)
    cs_ref[...] = cs
    mask_ref[...] = (cs > jnp.mean(cs)).astype(jnp.int32)
    cnt_ref[...] = cnt


def _epilogue(sums_ps, cnts_p):
    return pl.pallas_call(
        _epi_body,
        out_shape=[
            jax.ShapeDtypeStruct((BINS, 64), jnp.float32),
            jax.ShapeDtypeStruct((BINS,), jnp.float32),
            jax.ShapeDtypeStruct((BINS,), jnp.int32),
            jax.ShapeDtypeStruct((BINS,), jnp.float32),
        ],
    )(*sums_ps, cnts_p)


def kernel(feat, coords, n_x, n_y, n_z):
    n, d = feat.shape
    coords_t = coords.T
    feat_t = feat.T
    mins3, maxs3, xc, yc, zc = _coord_minmax(coords_t)
    mins = mins3[:, 0]
    maxs = maxs3[:, 0]
    extent = maxs - mins + 1e-6
    nf = jnp.stack([jnp.asarray(n_x), jnp.asarray(n_y), jnp.asarray(n_z)]
                   ).astype(jnp.float32)
    vals = jnp.concatenate([mins, extent, nf])
    params = jnp.zeros((16, 16), jnp.float32).at[:9, :].set(vals[:, None])
    zeros64 = jnp.zeros((BINS, 128), jnp.float32)
    zeros16 = jnp.zeros((BINS, 16), jnp.float32)
    ones = jnp.ones((IB, 16), jnp.float32)
    binid2, binid1, cnts_p = _sc_binning(params, xc, yc, zc, zeros16, ones)
    slab_n = n // NSLAB
    sums_ps = []
    for s in range(NSLAB):
        feat_slab = _transpose_slab(feat_t, s, slab_n)
        (sp,) = _sc_scatter(feat_slab, binid2, zeros64, s)
        sums_ps.append(sp)
    proto, cs, mask_i, counts = _epilogue(sums_ps, cnts_p)
    return proto, cs, mask_i.astype(bool), binid1, counts
